# baseline (speedup 1.0000x reference)
import functools

import jax
import jax.numpy as jnp
from jax import lax
from jax.experimental import pallas as pl
from jax.experimental.pallas import tpu as pltpu
from jax.experimental.pallas import tpu_sc as plsc

L = 16
NW = 32
NBINS = 1024
BMAX = 8.0
SCALE = NBINS / BMAX
SHIFT = 13
NPAIR = 56
NTASK = 224
QE = 65536
CHUNK = 4096
P = 262144
NROUND = NTASK // NW
HW = NBINS * L


def _sc_body(d_hbm, t_hbm, hist_out, accp_out, acci_out,
             lbuf, tbuf, hist, acc_v):
    wid = lax.axis_index("c") * 16 + lax.axis_index("s")
    lane = lax.iota(jnp.int32, L)
    zeros_i = jnp.zeros((L,), jnp.int32)
    zeros_v = jnp.zeros((L,), jnp.float32)

    @pl.loop(0, NROUND)
    def _round(r):
        t = r * NW + wid
        b = (t >> 2) & 7
        q = t & 3
        doff = t * QE
        toff = (b << 18) + (q << 16)

        @pl.loop(0, HW // L, unroll=8)
        def _zero(j):
            hist[pl.ds(j * L, L)] = zeros_i

        def _chunk(c, carry):
            accp, acci = carry
            do = pl.multiple_of(doff + c * CHUNK, CHUNK)
            to = pl.multiple_of(toff + c * CHUNK, CHUNK)
            pltpu.sync_copy(d_hbm.at[pl.ds(do, CHUNK)], lbuf)
            pltpu.sync_copy(t_hbm.at[pl.ds(to, CHUNK)], tbuf)

            def _vec(k, cr):
                ap, ai = cr
                sl = pl.ds(k * L, L)
                x = lbuf[sl]
                ti = tbuf[sl]
                tf = ti.astype(jnp.float32)
                s2 = x * tf
                e = 1.0 + (x - 2.0 * s2)
                bi = jnp.minimum(jnp.maximum(e * SCALE, 0.0),
                                 float(NBINS - 1)).astype(jnp.int32)
                idx = (bi << 4) + lane
                packed = (ti << SHIFT) + 1
                plsc.addupdate_scatter(hist, [idx], packed)
                prob = 1.0 / (1.0 + jnp.exp(-x))
                return (ap + prob, ai + prob * tf)

            return lax.fori_loop(0, CHUNK // L, _vec, (accp, acci), unroll=8)

        accp, acci = lax.fori_loop(0, QE // CHUNK, _chunk, (zeros_v, zeros_v))

        acc_v[pl.ds(0, L)] = accp
        acc_v[pl.ds(L, L)] = acci
        hoff = pl.multiple_of(t * HW, HW)
        aoff = pl.multiple_of(t * L, L)
        pltpu.sync_copy(hist, hist_out.at[pl.ds(hoff, HW)])
        pltpu.sync_copy(acc_v.at[pl.ds(0, L)], accp_out.at[pl.ds(aoff, L)])
        pltpu.sync_copy(acc_v.at[pl.ds(L, L)], acci_out.at[pl.ds(aoff, L)])


def _run_sc(dflat, tflat):
    f32 = jnp.float32
    mesh = plsc.VectorSubcoreMesh(core_axis_name="c", subcore_axis_name="s",
                                  num_cores=2, num_subcores=16)
    out_type = (
        jax.ShapeDtypeStruct((NTASK * HW,), jnp.int32),
        jax.ShapeDtypeStruct((NTASK * L,), f32),
        jax.ShapeDtypeStruct((NTASK * L,), f32),
    )
    scratch = [
        pltpu.VMEM((CHUNK,), f32),
        pltpu.VMEM((CHUNK,), jnp.int32),
        pltpu.VMEM((HW,), jnp.int32),
        pltpu.VMEM((2 * L,), f32),
    ]
    fn = pl.kernel(_sc_body, out_type=out_type, mesh=mesh,
                   scratch_types=scratch,
                   compiler_params=pltpu.CompilerParams(
                       needs_layout_passes=False))
    return fn(dflat, tflat)


B0_ROWS = NTASK * HW // 128
B0_GRID = 8
B0_BLK = B0_ROWS // B0_GRID


def _b0_body(x_ref, cnt_ref, pos_ref):
    x = x_ref[...]
    cnt16 = jnp.bitwise_and(x, (1 << SHIFT) - 1).astype(jnp.float32)
    pos16 = lax.shift_right_logical(x, SHIFT).astype(jnp.float32)
    j = lax.broadcasted_iota(jnp.int32, (128, 8), 0)
    k = lax.broadcasted_iota(jnp.int32, (128, 8), 1)
    m = (j >> 4 == k).astype(jnp.float32)
    cnt_ref[...] = jax.lax.dot(cnt16, m)
    pos_ref[...] = jax.lax.dot(pos16, m)


def _run_b0(hist):
    return pl.pallas_call(
        _b0_body,
        grid=(B0_GRID,),
        in_specs=[pl.BlockSpec((B0_BLK, 128), lambda i: (i, 0))],
        out_specs=[pl.BlockSpec((B0_BLK, 8), lambda i: (i, 0))] * 2,
        out_shape=[jax.ShapeDtypeStruct((B0_ROWS, 8), jnp.float32)] * 2,
    )(hist)


def _fold4(x):
    return (x[:, 0:NBINS] + x[:, NBINS:2 * NBINS]
            + x[:, 2 * NBINS:3 * NBINS] + x[:, 3 * NBINS:4 * NBINS])


def _phaseb_body(cnt_ref, pos_ref, accp_ref, acci_ref, out_ref):
    cnt = _fold4(cnt_ref[...])
    pos = _fold4(pos_ref[...])
    neg = cnt - pos
    centers = (lax.broadcasted_iota(jnp.int32, (NPAIR, NBINS), 1)
               .astype(jnp.float32) + 0.5) * (1.0 / SCALE)
    s = cnt * centers

    row = lax.broadcasted_iota(jnp.int32, (NBINS, NBINS), 0)
    col = lax.broadcasted_iota(jnp.int32, (NBINS, NBINS), 1)
    tri = (row <= col).astype(jnp.float32)
    A = jax.lax.dot(pos, tri)
    Bn = jax.lax.dot(neg, tri)
    G = A[:, NBINS - 1:NBINS]
    Nt = Bn[:, NBINS - 1:NBINS]
    n_hi = Nt - Bn
    gn = G + n_hi
    num = A * neg + pos * gn
    den = gn * (gn + neg)
    dj = jnp.where(den > 0.0, num / jnp.maximum(den, 1.0),
                   jnp.where(neg > 0.0, 1.0, 0.0))
    contrib = jnp.where(cnt > 0.0, s * dj / jnp.maximum(cnt, 1.0), 0.0)
    lov_pair = contrib.sum(axis=1, keepdims=True)
    sel_r = lax.broadcasted_iota(jnp.int32, (7, NPAIR), 0)
    sel_c = lax.broadcasted_iota(jnp.int32, (7, NPAIR), 1)
    sel = jnp.where(sel_c // 8 == sel_r, 0.125, 0.0)
    lov_i = jax.lax.dot(sel, lov_pair)

    tsum = jnp.sum(G[0:8, :])
    p_i = accp_ref[...].sum(axis=1, keepdims=True)
    i_i = acci_ref[...].sum(axis=1, keepdims=True)
    dice = 1.0 - (2.0 * i_i + 1.0) / (p_i + tsum + 1.0)

    w = jnp.where(
        lax.broadcasted_iota(jnp.int32, (7, 1), 0) == 0, 2.0, 1.0)
    out_ref[0, 0] = jnp.sum(w * (lov_i + dice))


def _run_phaseb(cnt, pos, accp, acci):
    return pl.pallas_call(
        _phaseb_body,
        out_shape=jax.ShapeDtypeStruct((1, 1), jnp.float32),
        in_specs=[pl.BlockSpec(memory_space=pltpu.VMEM)] * 4,
        out_specs=pl.BlockSpec(memory_space=pltpu.SMEM),
    )(cnt, pos, accp, acci)


def kernel(d0, d1, d2, d3, d4, d5, d6, target):
    dflat = jnp.stack([d0, d1, d2, d3, d4, d5, d6]).reshape(7 * 8 * P)
    tflat = target.reshape(8 * P)
    hist, accp, acci = _run_sc(dflat, tflat)
    cnt, pos = _run_b0(hist.reshape(B0_ROWS, 128))
    out = _run_phaseb(cnt.reshape(NPAIR, 4 * NBINS),
                      pos.reshape(NPAIR, 4 * NBINS),
                      accp.reshape(7, 8 * 4 * L), acci.reshape(7, 8 * 4 * L))
    return out[0, 0]
